# baseline (speedup 1.0000x reference)
import functools

import jax
import jax.numpy as jnp
from jax import lax
from jax.experimental import pallas as pl
from jax.experimental.pallas import tpu as pltpu
from jax.experimental.pallas import tpu_sc as plsc

_COL_GROUPS = 8
_ROW_GROUPS = 2
_BLOCK_ROWS = 2048


def _bag_body(L, w_hbm, part_hbm, w_v, out_v):
    wid = lax.axis_index("s")
    colg = wid % _COL_GROUPS
    rowg = wid // _COL_GROUPS
    rows = L // _ROW_GROUPS
    c0 = pl.multiple_of(colg * 128, 128)
    r0 = pl.multiple_of(rowg * rows, 8)

    pltpu.sync_copy(w_hbm.at[pl.ds(r0, rows), pl.ds(c0, 128)], w_v)

    scale = jnp.float32(1.0 / L)
    zero = jnp.zeros((16,), jnp.float32)

    def body(i, accs):
        return tuple(accs[v] + w_v[i, pl.ds(v * 16, 16)] for v in range(8))

    accs = lax.fori_loop(0, rows, body, (zero,) * 8)
    for v in range(8):
        out_v[pl.ds(v * 16, 16)] = accs[v] * scale

    pltpu.sync_copy(out_v, part_hbm.at[pl.ds(rowg * L + c0, 128)])


def _sc_partials(W, L):
    mesh = plsc.VectorSubcoreMesh(core_axis_name="c", subcore_axis_name="s",
                                  num_cores=1)
    return pl.kernel(
        functools.partial(_bag_body, L),
        out_type=jax.ShapeDtypeStruct((_ROW_GROUPS * L,), jnp.float32),
        mesh=mesh,
        scratch_types=[
            pltpu.VMEM((L // _ROW_GROUPS, 128), jnp.float32),
            pltpu.VMEM((128,), jnp.float32),
        ],
    )(W)


def _add_body(L, x_ref, part_ref, o_ref):
    bag = part_ref[pl.ds(0, L)] + part_ref[pl.ds(L, L)]
    o_ref[...] = x_ref[...] + bag.reshape(1, L)


def _tc_add(x2d, partials):
    rows, dim = x2d.shape
    grid = (rows // _BLOCK_ROWS,)
    return pl.pallas_call(
        functools.partial(_add_body, dim),
        grid=grid,
        in_specs=[
            pl.BlockSpec((_BLOCK_ROWS, dim), lambda i: (i, 0)),
            pl.BlockSpec((_ROW_GROUPS * dim,), lambda i: (0,)),
        ],
        out_specs=pl.BlockSpec((_BLOCK_ROWS, dim), lambda i: (i, 0)),
        out_shape=jax.ShapeDtypeStruct((rows, dim), jnp.float32),
        compiler_params=pltpu.CompilerParams(
            dimension_semantics=("parallel",),
        ),
    )(x2d, partials)


def kernel(x, W):
    B, S, L = x.shape
    partials = _sc_partials(W, L)
    x2d = x.reshape(B * S, L)
    out = _tc_add(x2d, partials)
    return out.reshape(B, S, L)


# speedup vs baseline: 1.0237x; 1.0237x over previous
import functools

import jax
import jax.numpy as jnp
from jax import lax
from jax.experimental import pallas as pl
from jax.experimental.pallas import tpu as pltpu
from jax.experimental.pallas import tpu_sc as plsc

_COL_GROUPS = 8
_ROW_GROUPS = 4
_BLOCK_ROWS = 2048


def _bag_body(L, w_hbm, part_hbm, w_v, out_v):
    core = lax.axis_index("c")
    sub = lax.axis_index("s")
    wid = sub * 2 + core
    colg = wid % _COL_GROUPS
    rowg = wid // _COL_GROUPS
    rows = L // _ROW_GROUPS
    c0 = pl.multiple_of(colg * 128, 128)
    r0 = pl.multiple_of(rowg * rows, 8)

    pltpu.sync_copy(w_hbm.at[pl.ds(r0, rows), pl.ds(c0, 128)], w_v)

    scale = jnp.float32(1.0 / L)
    zero = jnp.zeros((16,), jnp.float32)

    def body(i, accs):
        return tuple(accs[v] + w_v[i, pl.ds(v * 16, 16)] for v in range(8))

    accs = lax.fori_loop(0, rows, body, (zero,) * 8)
    for v in range(8):
        out_v[pl.ds(v * 16, 16)] = accs[v] * scale

    pltpu.sync_copy(out_v, part_hbm.at[pl.ds(rowg * L + c0, 128)])


def _sc_partials(W, L):
    mesh = plsc.VectorSubcoreMesh(core_axis_name="c", subcore_axis_name="s")
    return pl.kernel(
        functools.partial(_bag_body, L),
        out_type=jax.ShapeDtypeStruct((_ROW_GROUPS * L,), jnp.float32),
        mesh=mesh,
        scratch_types=[
            pltpu.VMEM((L // _ROW_GROUPS, 128), jnp.float32),
            pltpu.VMEM((128,), jnp.float32),
        ],
    )(W)


def _add_body(L, x_ref, part_ref, o_ref):
    bag = (part_ref[pl.ds(0, L)] + part_ref[pl.ds(L, L)]
           + part_ref[pl.ds(2 * L, L)] + part_ref[pl.ds(3 * L, L)])
    o_ref[...] = x_ref[...] + bag.reshape(1, L)


def _tc_add(x2d, partials):
    rows, dim = x2d.shape
    grid = (rows // _BLOCK_ROWS,)
    return pl.pallas_call(
        functools.partial(_add_body, dim),
        grid=grid,
        in_specs=[
            pl.BlockSpec((_BLOCK_ROWS, dim), lambda i: (i, 0)),
            pl.BlockSpec((_ROW_GROUPS * dim,), lambda i: (0,)),
        ],
        out_specs=pl.BlockSpec((_BLOCK_ROWS, dim), lambda i: (i, 0)),
        out_shape=jax.ShapeDtypeStruct((rows, dim), jnp.float32),
        compiler_params=pltpu.CompilerParams(
            dimension_semantics=("parallel",),
        ),
    )(x2d, partials)


def kernel(x, W):
    B, S, L = x.shape
    partials = _sc_partials(W, L)
    x2d = x.reshape(B * S, L)
    out = _tc_add(x2d, partials)
    return out.reshape(B, S, L)
